# Initial kernel scaffold; baseline (speedup 1.0000x reference)
import jax
import jax.numpy as jnp
from jax.experimental import pallas as pl


def kernel(features, edge_index, W, b):
    raise NotImplementedError("write your pallas kernel here")


# speedup vs baseline: 9.6104x; 9.6104x over previous
import functools

import jax
import jax.numpy as jnp
from jax import lax
from jax.experimental import pallas as pl
from jax.experimental.pallas import tpu as pltpu
from jax.experimental.pallas import tpu_sc as plsc

NC = 2
NS = 16
NW = NC * NS
CHUNK = 128
N_PAD = 10240
DUMMY = 10100
MM_BLOCK = 512


def _vs_mesh():
    return plsc.VectorSubcoreMesh(
        core_axis_name="c", subcore_axis_name="s", num_cores=NC, num_subcores=NS
    )


DEG_W = 32


@functools.lru_cache(maxsize=None)
def _deg_kernel(k_chunks: int):
    rows = N_PAD // NS

    def body(dst_hbm, ones_hbm, zero_hbm, out_hbm, acc_sh, dst_v, ones_v, sem):
        c = lax.axis_index("c")
        s = lax.axis_index("s")
        w = s * NC + c
        pltpu.sync_copy(zero_hbm.at[pl.ds(s * rows, rows)],
                        acc_sh.at[pl.ds(s * rows, rows)])
        pltpu.sync_copy(ones_hbm, ones_v)
        pltpu.sync_copy(dst_hbm.at[w], dst_v)
        plsc.subcore_barrier()

        def step(j, carry):
            pltpu.async_copy(ones_v, acc_sh.at[dst_v.at[j]], sem, add=True).wait()
            return carry

        lax.fori_loop(0, k_chunks, step, 0)
        plsc.subcore_barrier()
        pltpu.sync_copy(acc_sh.at[pl.ds(s * rows, rows)],
                        out_hbm.at[c, pl.ds(s * rows, rows)])

    return pl.kernel(
        body,
        out_type=jax.ShapeDtypeStruct((NC, N_PAD, DEG_W), jnp.float32),
        compiler_params=pltpu.CompilerParams(use_tc_tiling_on_sc=False),
        mesh=_vs_mesh(),
        scratch_types=[
            pltpu.VMEM_SHARED((N_PAD, DEG_W), jnp.float32),
            pltpu.VMEM((k_chunks, CHUNK), jnp.int32),
            pltpu.VMEM((CHUNK, DEG_W), jnp.float32),
            pltpu.SemaphoreType.DMA,
        ],
    )


@functools.lru_cache(maxsize=None)
def _hop_kernel(k_chunks: int, width: int):
    rows = N_PAD // NS

    def body(g_hbm, src_hbm, dst_hbm, zero_hbm, out_hbm,
             acc_sh, src_v, dst_v, rows_v, sem):
        c = lax.axis_index("c")
        s = lax.axis_index("s")
        w = s * NC + c
        pltpu.sync_copy(zero_hbm.at[pl.ds(s * rows, rows)],
                        acc_sh.at[pl.ds(s * rows, rows)])
        pltpu.sync_copy(src_hbm.at[w], src_v)
        pltpu.sync_copy(dst_hbm.at[w], dst_v)
        plsc.subcore_barrier()

        def step(j, carry):
            pltpu.async_copy(g_hbm.at[src_v.at[j]], rows_v, sem).wait()
            pltpu.async_copy(rows_v, acc_sh.at[dst_v.at[j]], sem, add=True).wait()
            return carry

        lax.fori_loop(0, k_chunks, step, 0)
        plsc.subcore_barrier()
        pltpu.sync_copy(acc_sh.at[pl.ds(s * rows, rows)],
                        out_hbm.at[c, pl.ds(s * rows, rows)])

    return pl.kernel(
        body,
        out_type=jax.ShapeDtypeStruct((NC, N_PAD, width), jnp.float32),
        compiler_params=pltpu.CompilerParams(use_tc_tiling_on_sc=False),
        mesh=_vs_mesh(),
        scratch_types=[
            pltpu.VMEM_SHARED((N_PAD, width), jnp.float32),
            pltpu.VMEM((k_chunks, CHUNK), jnp.int32),
            pltpu.VMEM((k_chunks, CHUNK), jnp.int32),
            pltpu.VMEM((CHUNK, width), jnp.float32),
            pltpu.SemaphoreType.DMA,
        ],
    )


def _mm(xp, w):
    f = xp.shape[1]
    cdim = w.shape[1]

    def body(x_ref, w_ref, o_ref):
        o_ref[...] = jnp.dot(x_ref[...], w_ref[...],
                             preferred_element_type=jnp.float32,
                             precision=lax.Precision.HIGHEST)

    return pl.pallas_call(
        body,
        grid=(N_PAD // MM_BLOCK,),
        in_specs=[
            pl.BlockSpec((MM_BLOCK, f), lambda i: (i, 0)),
            pl.BlockSpec((f, cdim), lambda i: (0, 0)),
        ],
        out_specs=pl.BlockSpec((MM_BLOCK, cdim), lambda i: (i, 0)),
        out_shape=jax.ShapeDtypeStruct((N_PAD, cdim), jnp.float32),
    )(xp, w)


def _norm(degp, n_nodes, cdim):
    def body(p_ref, nb_ref, n2_ref):
        dsum = jnp.sum(p_ref[0, :, :1] + p_ref[1, :, :1], axis=1, keepdims=True)
        row = lax.broadcasted_iota(jnp.int32, (N_PAD, 1), 0)
        nrm = jnp.where(row < n_nodes, lax.rsqrt(jnp.maximum(dsum, 1.0)), 0.0)
        nb_ref[...] = jnp.broadcast_to(nrm, (N_PAD, cdim))
        n2_ref[...] = jnp.broadcast_to(nrm * nrm, (N_PAD, cdim))

    return pl.pallas_call(
        body,
        out_shape=[
            jax.ShapeDtypeStruct((N_PAD, cdim), jnp.float32),
            jax.ShapeDtypeStruct((N_PAD, cdim), jnp.float32),
        ],
    )(degp)


def _scale(svec, a, c_arr, bias):
    cdim = a.shape[1]

    def body(s_ref, a_ref, c_ref, b_ref, o_ref):
        o_ref[...] = s_ref[...] * (a_ref[...] + c_ref[...]) + b_ref[...]

    return pl.pallas_call(
        body,
        out_shape=jax.ShapeDtypeStruct((N_PAD, cdim), jnp.float32),
    )(svec, a, c_arr, bias)


def kernel(features, edge_index, W, b):
    n, _ = features.shape
    cdim = W.shape[1]
    e = edge_index.shape[1]
    k_chunks = -(-e // (NW * CHUNK))
    e_pad = NW * CHUNK * k_chunks

    src = edge_index[0]
    dst = edge_index[1]
    fill = jnp.full((e_pad - e,), DUMMY, jnp.int32)
    src3 = jnp.concatenate([src, fill]).reshape(NW, k_chunks, CHUNK)
    dst3 = jnp.concatenate([dst, fill]).reshape(NW, k_chunks, CHUNK)

    xp = jnp.pad(features, ((0, N_PAD - n), (0, 0)))
    zeros_w = jnp.zeros((N_PAD, cdim), jnp.float32)
    zeros_8 = jnp.zeros((N_PAD, DEG_W), jnp.float32)
    ones_8 = jnp.ones((CHUNK, DEG_W), jnp.float32)
    bias_z = jnp.zeros((1, cdim), jnp.float32)

    degp = _deg_kernel(k_chunks)(dst3, ones_8, zeros_8)
    y = _mm(xp, W)
    norm_b, norm2_b = _norm(degp, n, cdim)
    g0 = _scale(norm_b, y, zeros_w, bias_z)
    p1 = _hop_kernel(k_chunks, cdim)(g0, src3, dst3, zeros_w)
    g1 = _scale(norm2_b, p1[0], p1[1], bias_z)
    p2 = _hop_kernel(k_chunks, cdim)(g1, src3, dst3, zeros_w)
    out = _scale(norm_b, p2[0], p2[1], jnp.reshape(b, (1, cdim)))
    return out[:n]
